# baseline (device time: 57614 ns/iter reference)
import jax
import jax.numpy as jnp
from jax import lax
from jax.experimental import pallas as pl
from jax.experimental.pallas import tpu as pltpu

N_DEV = 4
N_GLOBAL = 8192
EPS = 1e-5
BLK = 512
BARRIER = False
CHUNK_BLKS = 4
COMM = True


def kernel(x, gamma):
    m, n_local = x.shape
    n_blocks = m // BLK
    gamma2d = gamma.reshape(1, n_local)

    def body(x_hbm, gamma_ref, dummy_hbm, out_hbm, xbuf, xcache, p_col, p_row,
             comm_row, copy_sems, out_sems, send_sems, recv_sems):
        my = lax.axis_index("i")

        if BARRIER:
            barrier_sem = pltpu.get_barrier_semaphore()
            for o in range(1, N_DEV):
                pl.semaphore_signal(
                    barrier_sem, inc=1,
                    device_id=(lax.rem(my + o, N_DEV),),
                    device_id_type=pl.DeviceIdType.MESH,
                )
            pl.semaphore_wait(barrier_sem, N_DEV - 1)

        def in_copy(i, slot):
            return pltpu.make_async_copy(
                x_hbm.at[pl.ds(i * BLK, BLK), :],
                xbuf.at[slot],
                copy_sems.at[slot],
            )

        e11 = jnp.ones((1, 1), jnp.float32)
        crows = CHUNK_BLKS * BLK
        n_chunks = n_blocks // CHUNK_BLKS
        rdmas = []

        def send_chunk(c):
            csl = pl.ds(c * crows, crows)
            p_row[:, csl] = lax.dot_general(
                e11, p_col[csl, :], (((1,), (1,)), ((), ())),
                preferred_element_type=jnp.float32)
            if not COMM:
                return
            hs = []
            for o in range(1, N_DEV):
                rdma = pltpu.make_async_remote_copy(
                    src_ref=p_row.at[:, csl],
                    dst_ref=comm_row.at[o - 1, :, csl],
                    send_sem=send_sems.at[o - 1, c],
                    recv_sem=recv_sems.at[o - 1, c],
                    device_id=(lax.rem(my + o, N_DEV),),
                    device_id_type=pl.DeviceIdType.MESH,
                )
                rdma.start()
                hs.append(rdma)
            rdmas.append(hs)

        g = gamma_ref[:, :]
        in_copy(0, 0).start()
        in_copy(1, 1).start()
        for i in range(n_blocks):
            slot = i % 3
            in_copy(i, slot).wait()
            if i + 2 < n_blocks:
                in_copy(i + 2, (i + 2) % 3).start()
            xb = xbuf[slot]
            p_col[pl.ds(i * BLK, BLK), :] = jnp.sum(
                xb * xb, axis=1, keepdims=True)
            xcache[pl.ds(i * BLK, BLK), :] = (g * xb).astype(jnp.bfloat16)
            if (i + 1) % CHUNK_BLKS == 0:
                send_chunk(i // CHUNK_BLKS)

        out_dmas = [None] * n_blocks
        for c in range(n_chunks):
            csl = pl.ds(c * crows, crows)
            if COMM:
                for rdma in rdmas[c]:
                    rdma.wait()
                total = (p_row[:, csl] + comm_row[0, :, csl] +
                         comm_row[1, :, csl] + comm_row[2, :, csl])
            else:
                total = p_row[:, csl] * 4.0
            inv_row = lax.rsqrt(total * (1.0 / N_GLOBAL) + EPS)
            p_col[csl, :] = lax.dot_general(
                inv_row, e11, (((0,), (0,)), ((), ())),
                preferred_element_type=jnp.float32)
            for i in range(c * CHUNK_BLKS, (c + 1) * CHUNK_BLKS):
                sl = pl.ds(i * BLK, BLK)
                inv_b = p_col[sl, :].astype(jnp.bfloat16)
                xcache[sl, :] = xcache[sl, :] * inv_b
                if i >= 2:
                    out_dmas[i - 2].wait()
                out_dmas[i] = pltpu.make_async_copy(
                    xcache.at[sl, :], out_hbm.at[sl, :], out_sems.at[i % 2])
                out_dmas[i].start()
        out_dmas[n_blocks - 2].wait()
        out_dmas[n_blocks - 1].wait()

    return pl.pallas_call(
        body,
        out_shape=jax.ShapeDtypeStruct((m, n_local), jnp.bfloat16),
        in_specs=[
            pl.BlockSpec(memory_space=pl.ANY),
            pl.BlockSpec(memory_space=pltpu.VMEM),
            pl.BlockSpec(memory_space=pl.ANY),
        ],
        out_specs=pl.BlockSpec(memory_space=pl.ANY),
        input_output_aliases={2: 0},
        scratch_shapes=[
            pltpu.VMEM((3, BLK, n_local), jnp.float32),
            pltpu.VMEM((m, n_local), jnp.bfloat16),
            pltpu.VMEM((m, 1), jnp.float32),
            pltpu.VMEM((1, m), jnp.float32),
            pltpu.VMEM((N_DEV - 1, 1, m), jnp.float32),
            pltpu.SemaphoreType.DMA((3,)),
            pltpu.SemaphoreType.DMA((2,)),
            pltpu.SemaphoreType.DMA(
                (N_DEV - 1, m // (CHUNK_BLKS * BLK))),
            pltpu.SemaphoreType.DMA(
                (N_DEV - 1, m // (CHUNK_BLKS * BLK))),
        ],
        compiler_params=pltpu.CompilerParams(
            collective_id=None,
            vmem_limit_bytes=60 * 1024 * 1024,
        ),
    )(x, gamma2d, jnp.zeros((m, n_local), jnp.bfloat16))


# device time: 49055 ns/iter; 1.1745x vs baseline; 1.1745x over previous
import jax
import jax.numpy as jnp
from jax import lax
from jax.experimental import pallas as pl
from jax.experimental.pallas import tpu as pltpu

N_DEV = 4
N_GLOBAL = 8192
EPS = 1e-5
BLK = 1024
CHUNK_BLKS = 2
COMM = True


def _phase_a(x, gamma2d):
    m, n_local = x.shape
    n_blocks = m // BLK
    n_chunks = n_blocks // CHUNK_BLKS

    def body(x_hbm, gamma_ref, xc_hbm, inv_ref, xbuf, stage, p_col, p_row,
             comm_row, copy_sems, out_sems, send_sems, recv_sems):
        my = lax.axis_index("i")

        if COMM:
            barrier_sem = pltpu.get_barrier_semaphore()
            for o in range(1, N_DEV):
                pl.semaphore_signal(
                    barrier_sem, inc=1,
                    device_id=(lax.rem(my + o, N_DEV),),
                    device_id_type=pl.DeviceIdType.MESH,
                )
            pl.semaphore_wait(barrier_sem, N_DEV - 1)

        def in_copy(i, slot):
            return pltpu.make_async_copy(
                x_hbm.at[pl.ds(i * BLK, BLK), :],
                xbuf.at[slot],
                copy_sems.at[slot],
            )

        e11 = jnp.ones((1, 1), jnp.float32)
        crows = CHUNK_BLKS * BLK
        rdmas = []

        def send_chunk(c):
            csl = pl.ds(c * crows, crows)
            p_row[:, csl] = lax.dot_general(
                e11, p_col[csl, :], (((1,), (1,)), ((), ())),
                preferred_element_type=jnp.float32)
            if not COMM:
                return
            hs = []
            for o in range(1, N_DEV):
                rdma = pltpu.make_async_remote_copy(
                    src_ref=p_row.at[:, csl],
                    dst_ref=comm_row.at[o - 1, :, csl],
                    send_sem=send_sems.at[o - 1, c],
                    recv_sem=recv_sems.at[o - 1, c],
                    device_id=(lax.rem(my + o, N_DEV),),
                    device_id_type=pl.DeviceIdType.MESH,
                )
                rdma.start()
                hs.append(rdma)
            rdmas.append(hs)

        g = gamma_ref[:, :]
        out_dmas = [None] * n_blocks
        in_copy(0, 0).start()
        in_copy(1, 1).start()
        for i in range(n_blocks):
            slot = i % 3
            in_copy(i, slot).wait()
            if i + 2 < n_blocks:
                in_copy(i + 2, (i + 2) % 3).start()
            xb = xbuf[slot]
            sl = pl.ds(i * BLK, BLK)
            p_col[sl, :] = jnp.sum(xb * xb, axis=1, keepdims=True)
            if i >= 2:
                out_dmas[i - 2].wait()
            stage[i % 2] = (g * xb).astype(jnp.bfloat16)
            out_dmas[i] = pltpu.make_async_copy(
                stage.at[i % 2], xc_hbm.at[sl, :], out_sems.at[i % 2])
            out_dmas[i].start()
            if (i + 1) % CHUNK_BLKS == 0:
                send_chunk(i // CHUNK_BLKS)

        for c in range(n_chunks):
            csl = pl.ds(c * crows, crows)
            if COMM:
                for rdma in rdmas[c]:
                    rdma.wait()
                total = (p_row[:, csl] + comm_row[0, :, csl] +
                         comm_row[1, :, csl] + comm_row[2, :, csl])
            else:
                total = p_row[:, csl] * 4.0
            inv_ref[:, csl] = lax.rsqrt(total * (1.0 / N_GLOBAL) + EPS)

        out_dmas[n_blocks - 2].wait()
        out_dmas[n_blocks - 1].wait()

    return pl.pallas_call(
        body,
        out_shape=[
            jax.ShapeDtypeStruct((m, n_local), jnp.bfloat16),
            jax.ShapeDtypeStruct((1, m), jnp.float32),
        ],
        in_specs=[
            pl.BlockSpec(memory_space=pl.ANY),
            pl.BlockSpec(memory_space=pltpu.VMEM),
        ],
        out_specs=[
            pl.BlockSpec(memory_space=pl.ANY),
            pl.BlockSpec(memory_space=pltpu.VMEM),
        ],
        scratch_shapes=[
            pltpu.VMEM((3, BLK, n_local), jnp.float32),
            pltpu.VMEM((2, BLK, n_local), jnp.bfloat16),
            pltpu.VMEM((m, 1), jnp.float32),
            pltpu.VMEM((1, m), jnp.float32),
            pltpu.VMEM((N_DEV - 1, 1, m), jnp.float32),
            pltpu.SemaphoreType.DMA((3,)),
            pltpu.SemaphoreType.DMA((2,)),
            pltpu.SemaphoreType.DMA(
                (N_DEV - 1, m // (CHUNK_BLKS * BLK))),
            pltpu.SemaphoreType.DMA(
                (N_DEV - 1, m // (CHUNK_BLKS * BLK))),
        ],
        compiler_params=pltpu.CompilerParams(
            collective_id=0 if COMM else None,
            vmem_limit_bytes=60 * 1024 * 1024,
        ),
    )(x, gamma2d)


def _phase_b(xcache, inv_row):
    m, n_local = xcache.shape
    n_blocks = m // BLK

    def body(xc_hbm, inv_ref, out_hbm, xbuf, stage, inv_col,
             copy_sems, out_sems):
        e11 = jnp.ones((1, 1), jnp.float32)
        inv_col[:, :] = lax.dot_general(
            inv_ref[:, :], e11, (((0,), (0,)), ((), ())),
            preferred_element_type=jnp.float32)

        def in_copy(i, slot):
            return pltpu.make_async_copy(
                xc_hbm.at[pl.ds(i * BLK, BLK), :],
                xbuf.at[slot],
                copy_sems.at[slot],
            )

        out_dmas = [None] * n_blocks
        in_copy(0, 0).start()
        in_copy(1, 1).start()
        for i in range(n_blocks):
            slot = i % 3
            in_copy(i, slot).wait()
            if i + 2 < n_blocks:
                in_copy(i + 2, (i + 2) % 3).start()
            sl = pl.ds(i * BLK, BLK)
            inv_b = inv_col[sl, :].astype(jnp.bfloat16)
            if i >= 2:
                out_dmas[i - 2].wait()
            stage[i % 2] = xbuf[slot] * inv_b
            out_dmas[i] = pltpu.make_async_copy(
                stage.at[i % 2], out_hbm.at[sl, :], out_sems.at[i % 2])
            out_dmas[i].start()
        out_dmas[n_blocks - 2].wait()
        out_dmas[n_blocks - 1].wait()

    return pl.pallas_call(
        body,
        out_shape=jax.ShapeDtypeStruct((m, n_local), jnp.bfloat16),
        in_specs=[
            pl.BlockSpec(memory_space=pl.ANY),
            pl.BlockSpec(memory_space=pltpu.VMEM),
        ],
        out_specs=pl.BlockSpec(memory_space=pl.ANY),
        scratch_shapes=[
            pltpu.VMEM((3, BLK, n_local), jnp.bfloat16),
            pltpu.VMEM((2, BLK, n_local), jnp.bfloat16),
            pltpu.VMEM((m, 1), jnp.float32),
            pltpu.SemaphoreType.DMA((3,)),
            pltpu.SemaphoreType.DMA((2,)),
        ],
        compiler_params=pltpu.CompilerParams(
            vmem_limit_bytes=60 * 1024 * 1024,
        ),
    )(xcache, inv_row)


def kernel(x, gamma):
    m, n_local = x.shape
    gamma2d = gamma.reshape(1, n_local)
    xcache, inv_row = _phase_a(x, gamma2d)
    return _phase_b(xcache, inv_row)


# device time: 46107 ns/iter; 1.2496x vs baseline; 1.0639x over previous
import jax
import jax.numpy as jnp
from jax import lax
from jax.experimental import pallas as pl
from jax.experimental.pallas import tpu as pltpu

N_DEV = 4
N_GLOBAL = 8192
EPS = 1e-5
BLK = 512
BARRIER = True
CHUNK_BLKS = 4
COMM = True


def kernel(x, gamma):
    m, n_local = x.shape
    n_blocks = m // BLK
    gamma2d = gamma.reshape(1, n_local)

    def body(x_hbm, gamma_ref, out_hbm, xbuf, xcache, p_col, p_row, comm_row,
             copy_sems, out_sems, send_sems, recv_sems):
        my = lax.axis_index("i")

        if COMM or BARRIER:
            barrier_sem = pltpu.get_barrier_semaphore()
            for o in range(1, N_DEV):
                pl.semaphore_signal(
                    barrier_sem, inc=1,
                    device_id=(lax.rem(my + o, N_DEV),),
                    device_id_type=pl.DeviceIdType.MESH,
                )
            pl.semaphore_wait(barrier_sem, N_DEV - 1)

        def in_copy(i, slot):
            return pltpu.make_async_copy(
                x_hbm.at[pl.ds(i * BLK, BLK), :],
                xbuf.at[slot],
                copy_sems.at[slot],
            )

        e11 = jnp.ones((1, 1), jnp.float32)
        crows = CHUNK_BLKS * BLK
        n_chunks = n_blocks // CHUNK_BLKS
        rdmas = []

        def send_chunk(c):
            csl = pl.ds(c * crows, crows)
            p_row[:, csl] = lax.dot_general(
                e11, p_col[csl, :], (((1,), (1,)), ((), ())),
                preferred_element_type=jnp.float32)
            if not COMM:
                return
            hs = []
            for o in range(1, N_DEV):
                rdma = pltpu.make_async_remote_copy(
                    src_ref=p_row.at[:, csl],
                    dst_ref=comm_row.at[o - 1, :, csl],
                    send_sem=send_sems.at[o - 1, c],
                    recv_sem=recv_sems.at[o - 1, c],
                    device_id=(lax.rem(my + o, N_DEV),),
                    device_id_type=pl.DeviceIdType.MESH,
                )
                rdma.start()
                hs.append(rdma)
            rdmas.append(hs)

        g = gamma_ref[:, :]
        in_copy(0, 0).start()
        in_copy(1, 1).start()
        for i in range(n_blocks):
            slot = i % 3
            in_copy(i, slot).wait()
            if i + 2 < n_blocks:
                in_copy(i + 2, (i + 2) % 3).start()
            xb = xbuf[slot]
            p_col[pl.ds(i * BLK, BLK), :] = jnp.sum(
                xb * xb, axis=1, keepdims=True)
            xcache[pl.ds(i * BLK, BLK), :] = (g * xb).astype(jnp.bfloat16)
            if (i + 1) % CHUNK_BLKS == 0:
                send_chunk(i // CHUNK_BLKS)

        out_dmas = [None] * n_blocks
        for c in range(n_chunks):
            csl = pl.ds(c * crows, crows)
            if COMM:
                for rdma in rdmas[c]:
                    rdma.wait()
                total = (p_row[:, csl] + comm_row[0, :, csl] +
                         comm_row[1, :, csl] + comm_row[2, :, csl])
            else:
                total = p_row[:, csl] * 4.0
            inv_row = lax.rsqrt(total * (1.0 / N_GLOBAL) + EPS)
            p_col[csl, :] = lax.dot_general(
                inv_row, e11, (((0,), (0,)), ((), ())),
                preferred_element_type=jnp.float32)
            for i in range(c * CHUNK_BLKS, (c + 1) * CHUNK_BLKS):
                sl = pl.ds(i * BLK, BLK)
                inv_b = p_col[sl, :].astype(jnp.bfloat16)
                xcache[sl, :] = xcache[sl, :] * inv_b
                if i >= 2:
                    out_dmas[i - 2].wait()
                out_dmas[i] = pltpu.make_async_copy(
                    xcache.at[sl, :], out_hbm.at[sl, :], out_sems.at[i % 2])
                out_dmas[i].start()
        out_dmas[n_blocks - 2].wait()
        out_dmas[n_blocks - 1].wait()

    return pl.pallas_call(
        body,
        out_shape=jax.ShapeDtypeStruct((m, n_local), jnp.bfloat16),
        in_specs=[
            pl.BlockSpec(memory_space=pl.ANY),
            pl.BlockSpec(memory_space=pltpu.VMEM),
        ],
        out_specs=pl.BlockSpec(memory_space=pl.ANY),
        scratch_shapes=[
            pltpu.VMEM((3, BLK, n_local), jnp.float32),
            pltpu.VMEM((m, n_local), jnp.bfloat16),
            pltpu.VMEM((m, 1), jnp.float32),
            pltpu.VMEM((1, m), jnp.float32),
            pltpu.VMEM((N_DEV - 1, 1, m), jnp.float32),
            pltpu.SemaphoreType.DMA((3,)),
            pltpu.SemaphoreType.DMA((2,)),
            pltpu.SemaphoreType.DMA(
                (N_DEV - 1, m // (CHUNK_BLKS * BLK))),
            pltpu.SemaphoreType.DMA(
                (N_DEV - 1, m // (CHUNK_BLKS * BLK))),
        ],
        compiler_params=pltpu.CompilerParams(
            collective_id=0 if (COMM or BARRIER) else None,
            vmem_limit_bytes=60 * 1024 * 1024,
        ),
    )(x, gamma2d)
